# baseline (speedup 1.0000x reference)
import jax
import jax.numpy as jnp
from jax.experimental import pallas as pl
from jax.experimental.pallas import tpu as pltpu

_D_IN = 128
_D_HID = 32

_ROW_BLOCK = 5000


def _fused_dcrnn_kernel(x_ref, w_ref, bias_ref, wlin_ref, blin_ref, out_ref):
    xb = x_ref[...]
    g = jnp.dot(xb, w_ref[...], preferred_element_type=jnp.float32)
    t = jnp.tanh(g + bias_ref[...])
    h = jnp.maximum((1.0 - t[:, :_D_HID]) * t[:, _D_HID:], 0.0)
    out_ref[...] = (
        jnp.dot(h, wlin_ref[...], preferred_element_type=jnp.float32)
        + blin_ref[...])


def kernel(x, edge_index, edge_weight, W_z, b_z, W_r, b_r, W_h, b_h, W_lin,
           b_lin):
    del edge_index, edge_weight, W_r, b_r
    n = x.shape[0]
    az = 0.5 * (W_z[0, 0, :_D_IN, :] + W_z[1, 0, :_D_IN, :])
    ah = W_h[0, 0, :_D_IN, :] + W_h[1, 0, :_D_IN, :]
    azah = jnp.concatenate([az, ah], axis=1)
    bias = jnp.concatenate([0.5 * b_z, b_h]).reshape(1, 2 * _D_HID)
    wlin = 0.5 * W_lin
    blin = b_lin.reshape(1, -1)
    out_len = W_lin.shape[1]

    grid = (pl.cdiv(n, _ROW_BLOCK),)
    return pl.pallas_call(
        _fused_dcrnn_kernel,
        grid=grid,
        in_specs=[
            pl.BlockSpec((_ROW_BLOCK, _D_IN), lambda i: (i, 0)),
            pl.BlockSpec(azah.shape, lambda i: (0, 0)),
            pl.BlockSpec(bias.shape, lambda i: (0, 0)),
            pl.BlockSpec(wlin.shape, lambda i: (0, 0)),
            pl.BlockSpec(blin.shape, lambda i: (0, 0)),
        ],
        out_specs=pl.BlockSpec((_ROW_BLOCK, out_len), lambda i: (i, 0)),
        out_shape=jax.ShapeDtypeStruct((n, out_len), jnp.float32),
        compiler_params=pltpu.CompilerParams(
            dimension_semantics=("arbitrary",),
        ),
    )(x, azah, bias, wlin, blin)


# speedup vs baseline: 1.0825x; 1.0825x over previous
import jax
import jax.numpy as jnp
from jax.experimental import pallas as pl
from jax.experimental.pallas import tpu as pltpu

_D_IN = 128
_D_HID = 32

_ROW_BLOCK = 5000


def _fused_dcrnn_kernel(x_ref, wz_ref, bz_ref, wh_ref, bh_ref, wlin_ref,
                        blin_ref, out_ref):
    xb = x_ref[...]
    az = 0.5 * (wz_ref[0, :_D_IN, :] + wz_ref[1, :_D_IN, :])
    ah = wh_ref[0, :_D_IN, :] + wh_ref[1, :_D_IN, :]
    azah = jnp.concatenate([az, ah], axis=1)
    bias = jnp.concatenate([0.5 * bz_ref[...], bh_ref[...]], axis=1)
    g = jnp.dot(xb, azah, preferred_element_type=jnp.float32) + bias
    t = jnp.tanh(g)
    h = jnp.maximum((1.0 - t[:, :_D_HID]) * t[:, _D_HID:], 0.0)
    out_ref[...] = (
        jnp.dot(h, 0.5 * wlin_ref[...], preferred_element_type=jnp.float32)
        + blin_ref[...])


def kernel(x, edge_index, edge_weight, W_z, b_z, W_r, b_r, W_h, b_h, W_lin,
           b_lin):
    del edge_index, edge_weight, W_r, b_r
    n = x.shape[0]
    wz = W_z[:, 0]
    wh = W_h[:, 0]
    bz = b_z.reshape(1, _D_HID)
    bh = b_h.reshape(1, _D_HID)
    blin = b_lin.reshape(1, -1)
    out_len = W_lin.shape[1]

    grid = (pl.cdiv(n, _ROW_BLOCK),)
    return pl.pallas_call(
        _fused_dcrnn_kernel,
        grid=grid,
        in_specs=[
            pl.BlockSpec((_ROW_BLOCK, _D_IN), lambda i: (i, 0)),
            pl.BlockSpec(wz.shape, lambda i: (0, 0, 0)),
            pl.BlockSpec(bz.shape, lambda i: (0, 0)),
            pl.BlockSpec(wh.shape, lambda i: (0, 0, 0)),
            pl.BlockSpec(bh.shape, lambda i: (0, 0)),
            pl.BlockSpec(W_lin.shape, lambda i: (0, 0)),
            pl.BlockSpec(blin.shape, lambda i: (0, 0)),
        ],
        out_specs=pl.BlockSpec((_ROW_BLOCK, out_len), lambda i: (i, 0)),
        out_shape=jax.ShapeDtypeStruct((n, out_len), jnp.float32),
        compiler_params=pltpu.CompilerParams(
            dimension_semantics=("parallel",),
        ),
    )(x, wz, bz, wh, bh, W_lin, blin)
